# Initial kernel scaffold; baseline (speedup 1.0000x reference)
import jax
import jax.numpy as jnp
from jax.experimental import pallas as pl


def kernel(x, edge_index, edge_weight, W1, b1, W2, b2, Wout, bout):
    raise NotImplementedError("write your pallas kernel here")


# speedup vs baseline: 15.1756x; 15.1756x over previous
import functools

import jax
import jax.numpy as jnp
from jax import lax
from jax.experimental import pallas as pl
from jax.experimental.pallas import tpu as pltpu
from jax.experimental.pallas import tpu_sc as plsc

N_PAD = 10240
NC, NS, L = 2, 16, 16
NW = NC * NS
K = 128


def _mesh():
    return plsc.VectorSubcoreMesh(
        core_axis_name="c", subcore_axis_name="s", num_cores=NC, num_subcores=NS)


def _make_deg(n_chunks):
    @functools.partial(
        pl.kernel,
        out_type=jax.ShapeDtypeStruct((NC, N_PAD, L), jnp.float32),
        mesh=_mesh(),
        compiler_params=pltpu.CompilerParams(use_tc_tiling_on_sc=False),
        scratch_types=[
            pltpu.VMEM_SHARED((N_PAD, L), jnp.float32),
            pltpu.VMEM((K,), jnp.int32),
            pltpu.VMEM((K,), jnp.float32),
            pltpu.VMEM((K, L), jnp.float32),
        ],
    )
    def k(col_hbm, ew_hbm, zeros_hbm, out_hbm, acc, col_v, w_v, wbuf):
        cid = lax.axis_index("c")
        sid = lax.axis_index("s")
        rpt = N_PAD // NS
        pltpu.sync_copy(zeros_hbm.at[pl.ds(sid * rpt, rpt)],
                        acc.at[pl.ds(sid * rpt, rpt)])
        plsc.subcore_barrier()
        ept = n_chunks * K
        base = (cid * NS + sid) * ept

        def body(i, c):
            off = base + i * K
            pltpu.sync_copy(col_hbm.at[pl.ds(off, K)], col_v)
            pltpu.sync_copy(ew_hbm.at[pl.ds(off, K)], w_v)
            for g in range(K // L):
                w16 = w_v[pl.ds(g * L, L)]
                for j in range(L):
                    wbuf[g * L + j, pl.ds(0, L)] = jnp.full((L,), w16[j],
                                                            jnp.float32)
            pltpu.sync_copy(wbuf, acc.at[col_v], add=True)
            return c

        lax.fori_loop(0, n_chunks, body, 0)
        plsc.subcore_barrier()
        pltpu.sync_copy(acc.at[pl.ds(sid * rpt, rpt)],
                        out_hbm.at[cid, pl.ds(sid * rpt, rpt)])

    return k


def _make_agg(F, n_chunks):
    @functools.partial(
        pl.kernel,
        out_type=jax.ShapeDtypeStruct((NC, N_PAD, F), jnp.float32),
        mesh=_mesh(),
        compiler_params=pltpu.CompilerParams(use_tc_tiling_on_sc=False),
        scratch_types=[
            pltpu.VMEM_SHARED((N_PAD, F), jnp.float32),
            pltpu.VMEM((K,), jnp.int32),
            pltpu.VMEM((K,), jnp.int32),
            pltpu.VMEM((K,), jnp.float32),
            pltpu.VMEM((K, F), jnp.float32),
            pltpu.SemaphoreType.DMA,
        ],
    )
    def k(row_hbm, col_hbm, ew_hbm, g_hbm, zeros_hbm, out_hbm,
          acc, row_v, col_v, w_v, rows_v, sem):
        cid = lax.axis_index("c")
        sid = lax.axis_index("s")
        rpt = N_PAD // NS
        pltpu.sync_copy(zeros_hbm.at[pl.ds(sid * rpt, rpt)],
                        acc.at[pl.ds(sid * rpt, rpt)])
        plsc.subcore_barrier()
        ept = n_chunks * K
        base = (cid * NS + sid) * ept
        nv = F // L

        def body(i, c):
            off = base + i * K
            pltpu.sync_copy(row_hbm.at[pl.ds(off, K)], row_v)
            pltpu.sync_copy(col_hbm.at[pl.ds(off, K)], col_v)
            pltpu.sync_copy(ew_hbm.at[pl.ds(off, K)], w_v)
            pltpu.async_copy(g_hbm.at[row_v], rows_v, sem).wait()
            for g in range(K // L):
                w16 = w_v[pl.ds(g * L, L)]
                for j in range(L):
                    e = g * L + j
                    wj = w16[j]
                    for h in range(nv):
                        rows_v[e, pl.ds(h * L, L)] = rows_v[e, pl.ds(h * L, L)] * wj
            pltpu.sync_copy(rows_v, acc.at[col_v], add=True)
            return c

        lax.fori_loop(0, n_chunks, body, 0)
        plsc.subcore_barrier()
        pltpu.sync_copy(acc.at[pl.ds(sid * rpt, rpt)],
                        out_hbm.at[cid, pl.ds(sid * rpt, rpt)])

    return k


def _tc1(x, W1, degp):
    n, _ = x.shape
    h = W1.shape[1]

    def body(x_ref, w_ref, degp_ref, h_ref, g_ref, dinv_ref):
        deg = degp_ref[0, :, 0:1] + degp_ref[1, :, 0:1] + 1.0
        dinv_full = jnp.where(deg > 0, lax.rsqrt(jnp.maximum(deg, 1e-12)), 0.0)
        dinv = dinv_full[:n]
        hm = jnp.dot(x_ref[...], w_ref[...], preferred_element_type=jnp.float32)
        h_ref[...] = hm
        g_ref[...] = hm * dinv
        dinv_ref[...] = dinv

    return pl.pallas_call(
        body,
        out_shape=[
            jax.ShapeDtypeStruct((n, h), jnp.float32),
            jax.ShapeDtypeStruct((n, h), jnp.float32),
            jax.ShapeDtypeStruct((n, 1), jnp.float32),
        ],
    )(x, W1, degp)


def _tc2(accp, h, dinv, b, W2):
    n, _ = h.shape
    h2 = W2.shape[1]

    def body(accp_ref, h_ref, dinv_ref, b_ref, w_ref, hd2_ref, g2_ref):
        a = accp_ref[...]
        agg = a[0, :n] + a[1, :n]
        di = dinv_ref[...]
        pre = di * agg + (di * di) * h_ref[...] + b_ref[...]
        h1 = jnp.maximum(pre, 0.0)
        hd2 = jnp.dot(h1, w_ref[...], preferred_element_type=jnp.float32)
        hd2_ref[...] = hd2
        g2_ref[...] = hd2 * di

    return pl.pallas_call(
        body,
        out_shape=[
            jax.ShapeDtypeStruct((n, h2), jnp.float32),
            jax.ShapeDtypeStruct((n, h2), jnp.float32),
        ],
    )(accp, h, dinv, b, W2)


def _tc3(accp, hd2, dinv, b, Wout, bout):
    n, _ = hd2.shape

    def body(accp_ref, hd2_ref, dinv_ref, b_ref, w_ref, bout_ref, out_ref):
        a = accp_ref[...]
        agg = a[0, :n] + a[1, :n]
        di = dinv_ref[...]
        pre = di * agg + (di * di) * hd2_ref[...] + b_ref[...]
        h2 = jnp.maximum(pre, 0.0)
        z = jnp.dot(h2, w_ref[...], preferred_element_type=jnp.float32) + bout_ref[...]
        out_ref[...] = 1.0 / (1.0 + jnp.exp(-z))

    return pl.pallas_call(
        body,
        out_shape=jax.ShapeDtypeStruct((n, 1), jnp.float32),
    )(accp, hd2, dinv, b, Wout, bout)


def kernel(x, edge_index, edge_weight, W1, b1, W2, b2, Wout, bout):
    e_total = edge_weight.shape[0]
    row = edge_index[0].astype(jnp.int32)
    col = edge_index[1].astype(jnp.int32)
    ew = edge_weight.astype(jnp.float32)

    chunk_total = NW * K
    e_pad = ((e_total + chunk_total - 1) // chunk_total) * chunk_total
    pad = e_pad - e_total
    if pad:
        row = jnp.concatenate([row, jnp.zeros((pad,), jnp.int32)])
        col = jnp.concatenate([col, jnp.zeros((pad,), jnp.int32)])
        ew = jnp.concatenate([ew, jnp.zeros((pad,), jnp.float32)])
    n_chunks = e_pad // chunk_total

    h1n = W1.shape[1]
    h2n = W2.shape[1]
    zeros1 = jnp.zeros((N_PAD, L), jnp.float32)
    zeros_a = jnp.zeros((N_PAD, h1n), jnp.float32)
    zeros_b = jnp.zeros((N_PAD, h2n), jnp.float32)

    degp = _make_deg(n_chunks)(col, ew, zeros1)
    h, g1, dinv = _tc1(x, W1, degp)
    acc1 = _make_agg(h1n, n_chunks)(row, col, ew, g1, zeros_a)
    hd2, g2 = _tc2(acc1, h, dinv, b1.reshape(1, h1n), W2)
    acc2 = _make_agg(h2n, n_chunks)(row, col, ew, g2, zeros_b)
    return _tc3(acc2, hd2, dinv, b2.reshape(1, h2n), Wout, bout.reshape(1, 1))


# speedup vs baseline: 28.4193x; 1.8727x over previous
import functools

import jax
import jax.numpy as jnp
from jax import lax
from jax.experimental import pallas as pl
from jax.experimental.pallas import tpu as pltpu
from jax.experimental.pallas import tpu_sc as plsc

N_PAD = 10240
NC, NS, L = 2, 16, 16
NW = NC * NS
K = 128


def _mesh():
    return plsc.VectorSubcoreMesh(
        core_axis_name="c", subcore_axis_name="s", num_cores=NC, num_subcores=NS)


def _make_deg(n_chunks):
    assert n_chunks % 2 == 0

    @functools.partial(
        pl.kernel,
        out_type=jax.ShapeDtypeStruct((NC, N_PAD, L), jnp.float32),
        mesh=_mesh(),
        compiler_params=pltpu.CompilerParams(use_tc_tiling_on_sc=False),
        scratch_types=[
            pltpu.VMEM_SHARED((N_PAD, L), jnp.float32),
            pltpu.VMEM((n_chunks, K), jnp.int32),
            pltpu.VMEM((n_chunks, K), jnp.float32),
            pltpu.VMEM((K, L), jnp.float32),
            pltpu.VMEM((K, L), jnp.float32),
            pltpu.SemaphoreType.DMA,
            pltpu.SemaphoreType.DMA,
        ],
    )
    def k(col_hbm, ew_hbm, zeros_hbm, out_hbm,
          acc, col_b, w_b, wbuf0, wbuf1, ssem0, ssem1):
        cid = lax.axis_index("c")
        sid = lax.axis_index("s")
        wid = cid * NS + sid
        rpt = N_PAD // NS
        pltpu.sync_copy(zeros_hbm.at[pl.ds(sid * rpt, rpt)],
                        acc.at[pl.ds(sid * rpt, rpt)])
        pltpu.sync_copy(col_hbm.at[wid], col_b)
        pltpu.sync_copy(ew_hbm.at[wid], w_b)
        plsc.subcore_barrier()

        def build(wbuf, c):
            for g in range(K // L):
                w16 = w_b[c, pl.ds(g * L, L)]
                for j in range(L):
                    wbuf[g * L + j, pl.ds(0, L)] = jnp.full((L,), w16[j],
                                                            jnp.float32)

        def scatter(c, wbuf, sem):
            pltpu.async_copy(wbuf, acc.at[col_b.at[c]], sem, add=True)

        def wait_scatter(c, wbuf, sem):
            pltpu.make_async_copy(wbuf, acc.at[col_b.at[c]], sem).wait()

        def body(s, carry):
            c0 = 2 * s
            c1 = 2 * s + 1

            @pl.when(s > 0)
            def _():
                wait_scatter(c0 - 2, wbuf0, ssem0)

            build(wbuf0, c0)
            scatter(c0, wbuf0, ssem0)

            @pl.when(s > 0)
            def _():
                wait_scatter(c1 - 2, wbuf1, ssem1)

            build(wbuf1, c1)
            scatter(c1, wbuf1, ssem1)
            return carry

        lax.fori_loop(0, n_chunks // 2, body, 0)
        wait_scatter(n_chunks - 2, wbuf0, ssem0)
        wait_scatter(n_chunks - 1, wbuf1, ssem1)
        plsc.subcore_barrier()
        pltpu.sync_copy(acc.at[pl.ds(sid * rpt, rpt)],
                        out_hbm.at[cid, pl.ds(sid * rpt, rpt)])

    return k


def _make_agg(F, n_chunks):
    assert n_chunks % 2 == 0

    @functools.partial(
        pl.kernel,
        out_type=jax.ShapeDtypeStruct((NC, N_PAD, F), jnp.float32),
        mesh=_mesh(),
        compiler_params=pltpu.CompilerParams(use_tc_tiling_on_sc=False),
        scratch_types=[
            pltpu.VMEM_SHARED((N_PAD, F), jnp.float32),
            pltpu.VMEM((n_chunks, K), jnp.int32),
            pltpu.VMEM((n_chunks, K), jnp.int32),
            pltpu.VMEM((n_chunks, K), jnp.float32),
            pltpu.VMEM((K, F), jnp.float32),
            pltpu.VMEM((K, F), jnp.float32),
            pltpu.SemaphoreType.DMA,
            pltpu.SemaphoreType.DMA,
            pltpu.SemaphoreType.DMA,
            pltpu.SemaphoreType.DMA,
        ],
    )
    def k(row_hbm, col_hbm, ew_hbm, g_hbm, zeros_hbm, out_hbm,
          acc, row_b, col_b, w_b, rbuf0, rbuf1, gsem0, gsem1, ssem0, ssem1):
        cid = lax.axis_index("c")
        sid = lax.axis_index("s")
        wid = cid * NS + sid
        rpt = N_PAD // NS
        pltpu.sync_copy(zeros_hbm.at[pl.ds(sid * rpt, rpt)],
                        acc.at[pl.ds(sid * rpt, rpt)])
        pltpu.sync_copy(row_hbm.at[wid], row_b)
        pltpu.sync_copy(col_hbm.at[wid], col_b)
        pltpu.sync_copy(ew_hbm.at[wid], w_b)
        plsc.subcore_barrier()

        def mult(rbuf, c):
            for g in range(K // L):
                w16 = w_b[c, pl.ds(g * L, L)]
                for j in range(L):
                    e = g * L + j
                    wj = w16[j]
                    for h in range(F // L):
                        rbuf[e, pl.ds(h * L, L)] = rbuf[e, pl.ds(h * L, L)] * wj

        def gather(c, rbuf, sem):
            pltpu.async_copy(g_hbm.at[row_b.at[c]], rbuf, sem)

        def wait_gather(c, rbuf, sem):
            pltpu.make_async_copy(g_hbm.at[row_b.at[c]], rbuf, sem).wait()

        def scatter(c, rbuf, sem):
            pltpu.async_copy(rbuf, acc.at[col_b.at[c]], sem, add=True)

        def wait_scatter(c, rbuf, sem):
            pltpu.make_async_copy(rbuf, acc.at[col_b.at[c]], sem).wait()

        gather(0, rbuf0, gsem0)
        gather(1, rbuf1, gsem1)

        def body(s, carry):
            c0 = 2 * s
            c1 = 2 * s + 1
            wait_gather(c0, rbuf0, gsem0)
            mult(rbuf0, c0)
            scatter(c0, rbuf0, ssem0)
            wait_gather(c1, rbuf1, gsem1)
            mult(rbuf1, c1)
            scatter(c1, rbuf1, ssem1)

            @pl.when(c0 + 2 < n_chunks)
            def _():
                wait_scatter(c0, rbuf0, ssem0)
                gather(c0 + 2, rbuf0, gsem0)
                wait_scatter(c1, rbuf1, ssem1)
                gather(c1 + 2, rbuf1, gsem1)

            return carry

        lax.fori_loop(0, n_chunks // 2, body, 0)
        wait_scatter(n_chunks - 2, rbuf0, ssem0)
        wait_scatter(n_chunks - 1, rbuf1, ssem1)
        plsc.subcore_barrier()
        pltpu.sync_copy(acc.at[pl.ds(sid * rpt, rpt)],
                        out_hbm.at[cid, pl.ds(sid * rpt, rpt)])

    return k


def _tc1(x, W1, degp):
    n, _ = x.shape
    h = W1.shape[1]

    def body(x_ref, w_ref, degp_ref, h_ref, g_ref, dinv_ref):
        deg = degp_ref[0, :, 0:1] + degp_ref[1, :, 0:1] + 1.0
        dinv_full = jnp.where(deg > 0, lax.rsqrt(jnp.maximum(deg, 1e-12)), 0.0)
        dinv = dinv_full[:n]
        hm = jnp.dot(x_ref[...], w_ref[...], preferred_element_type=jnp.float32)
        h_ref[...] = hm
        g_ref[...] = hm * dinv
        dinv_ref[...] = dinv

    return pl.pallas_call(
        body,
        out_shape=[
            jax.ShapeDtypeStruct((n, h), jnp.float32),
            jax.ShapeDtypeStruct((n, h), jnp.float32),
            jax.ShapeDtypeStruct((n, 1), jnp.float32),
        ],
    )(x, W1, degp)


def _tc2(accp, h, dinv, b, W2):
    n, _ = h.shape
    h2 = W2.shape[1]

    def body(accp_ref, h_ref, dinv_ref, b_ref, w_ref, hd2_ref, g2_ref):
        a = accp_ref[...]
        agg = a[0, :n] + a[1, :n]
        di = dinv_ref[...]
        pre = di * agg + (di * di) * h_ref[...] + b_ref[...]
        h1 = jnp.maximum(pre, 0.0)
        hd2 = jnp.dot(h1, w_ref[...], preferred_element_type=jnp.float32)
        hd2_ref[...] = hd2
        g2_ref[...] = hd2 * di

    return pl.pallas_call(
        body,
        out_shape=[
            jax.ShapeDtypeStruct((n, h2), jnp.float32),
            jax.ShapeDtypeStruct((n, h2), jnp.float32),
        ],
    )(accp, h, dinv, b, W2)


def _tc3(accp, hd2, dinv, b, Wout, bout):
    n, _ = hd2.shape

    def body(accp_ref, hd2_ref, dinv_ref, b_ref, w_ref, bout_ref, out_ref):
        a = accp_ref[...]
        agg = a[0, :n] + a[1, :n]
        di = dinv_ref[...]
        pre = di * agg + (di * di) * hd2_ref[...] + b_ref[...]
        h2 = jnp.maximum(pre, 0.0)
        z = jnp.dot(h2, w_ref[...], preferred_element_type=jnp.float32) + bout_ref[...]
        out_ref[...] = 1.0 / (1.0 + jnp.exp(-z))

    return pl.pallas_call(
        body,
        out_shape=jax.ShapeDtypeStruct((n, 1), jnp.float32),
    )(accp, hd2, dinv, b, Wout, bout)


def kernel(x, edge_index, edge_weight, W1, b1, W2, b2, Wout, bout):
    e_total = edge_weight.shape[0]
    row = edge_index[0].astype(jnp.int32)
    col = edge_index[1].astype(jnp.int32)
    ew = edge_weight.astype(jnp.float32)

    chunk_total = NW * K * 2
    e_pad = ((e_total + chunk_total - 1) // chunk_total) * chunk_total
    pad = e_pad - e_total
    if pad:
        row = jnp.concatenate([row, jnp.zeros((pad,), jnp.int32)])
        col = jnp.concatenate([col, jnp.zeros((pad,), jnp.int32)])
        ew = jnp.concatenate([ew, jnp.zeros((pad,), jnp.float32)])
    n_chunks = e_pad // (NW * K)
    row = row.reshape(NW, n_chunks, K)
    col = col.reshape(NW, n_chunks, K)
    ew = ew.reshape(NW, n_chunks, K)

    h1n = W1.shape[1]
    h2n = W2.shape[1]
    zeros1 = jnp.zeros((N_PAD, L), jnp.float32)
    zeros_a = jnp.zeros((N_PAD, h1n), jnp.float32)
    zeros_b = jnp.zeros((N_PAD, h2n), jnp.float32)

    degp = _make_deg(n_chunks)(col, ew, zeros1)
    h, g1, dinv = _tc1(x, W1, degp)
    acc1 = _make_agg(h1n, n_chunks)(row, col, ew, g1, zeros_a)
    hd2, g2 = _tc2(acc1, h, dinv, b1.reshape(1, h1n), W2)
    acc2 = _make_agg(h2n, n_chunks)(row, col, ew, g2, zeros_b)
    return _tc3(acc2, hd2, dinv, b2.reshape(1, h2n), Wout, bout.reshape(1, 1))


# speedup vs baseline: 31.2276x; 1.0988x over previous
import functools

import jax
import jax.numpy as jnp
from jax import lax
from jax.experimental import pallas as pl
from jax.experimental.pallas import tpu as pltpu
from jax.experimental.pallas import tpu_sc as plsc

N_PAD = 10240
NC, NS, L = 2, 16, 16
NW = NC * NS
K = 128


def _mesh():
    return plsc.VectorSubcoreMesh(
        core_axis_name="c", subcore_axis_name="s", num_cores=NC, num_subcores=NS)


def _make_deg(n_chunks):
    assert n_chunks % 2 == 0

    @functools.partial(
        pl.kernel,
        out_type=jax.ShapeDtypeStruct((NC, N_PAD, L), jnp.float32),
        mesh=_mesh(),
        compiler_params=pltpu.CompilerParams(use_tc_tiling_on_sc=False),
        scratch_types=[
            pltpu.VMEM_SHARED((N_PAD, L), jnp.float32),
            pltpu.VMEM((n_chunks, K), jnp.int32),
            pltpu.VMEM((n_chunks, K), jnp.float32),
            pltpu.VMEM((K, L), jnp.float32),
            pltpu.VMEM((K, L), jnp.float32),
            pltpu.SemaphoreType.DMA,
            pltpu.SemaphoreType.DMA,
        ],
    )
    def k(col_hbm, ew_hbm, zeros_hbm, out_hbm,
          acc, col_b, w_b, wbuf0, wbuf1, ssem0, ssem1):
        cid = lax.axis_index("c")
        sid = lax.axis_index("s")
        wid = cid * NS + sid
        rpt = N_PAD // NS
        pltpu.sync_copy(zeros_hbm.at[pl.ds(sid * rpt, rpt)],
                        acc.at[pl.ds(sid * rpt, rpt)])
        pltpu.sync_copy(col_hbm.at[wid], col_b)
        pltpu.sync_copy(ew_hbm.at[wid], w_b)
        plsc.subcore_barrier()

        def build(wbuf, c):
            for g in range(K // L):
                w16 = w_b[c, pl.ds(g * L, L)]
                for j in range(L):
                    wbuf[g * L + j, pl.ds(0, L)] = jnp.full((L,), w16[j],
                                                            jnp.float32)

        def scatter(c, wbuf, sem):
            pltpu.async_copy(wbuf, acc.at[col_b.at[c]], sem, add=True)

        def wait_scatter(c, wbuf, sem):
            pltpu.make_async_copy(wbuf, acc.at[col_b.at[c]], sem).wait()

        def body(s, carry):
            c0 = 2 * s
            c1 = 2 * s + 1

            @pl.when(s > 0)
            def _():
                wait_scatter(c0 - 2, wbuf0, ssem0)

            build(wbuf0, c0)
            scatter(c0, wbuf0, ssem0)

            @pl.when(s > 0)
            def _():
                wait_scatter(c1 - 2, wbuf1, ssem1)

            build(wbuf1, c1)
            scatter(c1, wbuf1, ssem1)
            return carry

        lax.fori_loop(0, n_chunks // 2, body, 0)
        wait_scatter(n_chunks - 2, wbuf0, ssem0)
        wait_scatter(n_chunks - 1, wbuf1, ssem1)
        plsc.subcore_barrier()
        pltpu.sync_copy(acc.at[pl.ds(sid * rpt, rpt)],
                        out_hbm.at[cid, pl.ds(sid * rpt, rpt)])

    return k


def _make_agg(F, n_chunks):
    NB = 4
    assert n_chunks % NB == 0

    @functools.partial(
        pl.kernel,
        out_type=jax.ShapeDtypeStruct((NC, N_PAD, F), jnp.float32),
        mesh=_mesh(),
        compiler_params=pltpu.CompilerParams(use_tc_tiling_on_sc=False),
        scratch_types=[
            pltpu.VMEM_SHARED((N_PAD, F), jnp.float32),
            pltpu.VMEM((n_chunks, K), jnp.int32),
            pltpu.VMEM((n_chunks, K), jnp.int32),
            pltpu.VMEM((n_chunks, K), jnp.float32),
            [pltpu.VMEM((K, F), jnp.float32)] * NB,
            [pltpu.SemaphoreType.DMA] * NB,
            [pltpu.SemaphoreType.DMA] * NB,
        ],
    )
    def k(row_hbm, col_hbm, ew_hbm, g_hbm, zeros_hbm, out_hbm,
          acc, row_b, col_b, w_b, rbufs, gsems, ssems):
        cid = lax.axis_index("c")
        sid = lax.axis_index("s")
        wid = cid * NS + sid
        rpt = N_PAD // NS
        pltpu.sync_copy(zeros_hbm.at[pl.ds(sid * rpt, rpt)],
                        acc.at[pl.ds(sid * rpt, rpt)])
        pltpu.sync_copy(row_hbm.at[wid], row_b)
        pltpu.sync_copy(col_hbm.at[wid], col_b)
        pltpu.sync_copy(ew_hbm.at[wid], w_b)
        plsc.subcore_barrier()

        def mult(rbuf, c):
            for g in range(K // L):
                w16 = w_b[c, pl.ds(g * L, L)]
                for j in range(L):
                    e = g * L + j
                    wj = w16[j]
                    for h in range(F // L):
                        rbuf[e, pl.ds(h * L, L)] = rbuf[e, pl.ds(h * L, L)] * wj

        def gather(c, rbuf, sem):
            pltpu.async_copy(g_hbm.at[row_b.at[c]], rbuf, sem)

        def wait_gather(c, rbuf, sem):
            pltpu.make_async_copy(g_hbm.at[row_b.at[c]], rbuf, sem).wait()

        def scatter(c, rbuf, sem):
            pltpu.async_copy(rbuf, acc.at[col_b.at[c]], sem, add=True)

        def wait_scatter(c, rbuf, sem):
            pltpu.make_async_copy(rbuf, acc.at[col_b.at[c]], sem).wait()

        for t in range(NB - 1):
            gather(t, rbufs[t], gsems[t])

        def body(s, carry):
            for t in range(NB):
                c = NB * s + t
                wait_gather(c, rbufs[t], gsems[t])
                tp = (t + NB - 1) % NB

                @pl.when(c > 0)
                def _():
                    wait_scatter(c - 1, rbufs[tp], ssems[tp])

                @pl.when(c + NB - 1 < n_chunks)
                def _():
                    gather(c + NB - 1, rbufs[tp], gsems[tp])

                mult(rbufs[t], c)
                scatter(c, rbufs[t], ssems[t])
            return carry

        lax.fori_loop(0, n_chunks // NB, body, 0)
        wait_scatter(n_chunks - 1, rbufs[NB - 1], ssems[NB - 1])
        plsc.subcore_barrier()
        pltpu.sync_copy(acc.at[pl.ds(sid * rpt, rpt)],
                        out_hbm.at[cid, pl.ds(sid * rpt, rpt)])

    return k


def _tc1(x, W1, degp):
    n, _ = x.shape
    h = W1.shape[1]

    def body(x_ref, w_ref, degp_ref, h_ref, g_ref, dinv_ref):
        deg = degp_ref[0, :, 0:1] + degp_ref[1, :, 0:1] + 1.0
        dinv_full = jnp.where(deg > 0, lax.rsqrt(jnp.maximum(deg, 1e-12)), 0.0)
        dinv = dinv_full[:n]
        hm = jnp.dot(x_ref[...], w_ref[...], preferred_element_type=jnp.float32)
        h_ref[...] = hm
        g_ref[...] = hm * dinv
        dinv_ref[...] = dinv

    return pl.pallas_call(
        body,
        out_shape=[
            jax.ShapeDtypeStruct((n, h), jnp.float32),
            jax.ShapeDtypeStruct((n, h), jnp.float32),
            jax.ShapeDtypeStruct((n, 1), jnp.float32),
        ],
    )(x, W1, degp)


def _tc2(accp, h, dinv, b, W2):
    n, _ = h.shape
    h2 = W2.shape[1]

    def body(accp_ref, h_ref, dinv_ref, b_ref, w_ref, hd2_ref, g2_ref):
        a = accp_ref[...]
        agg = a[0, :n] + a[1, :n]
        di = dinv_ref[...]
        pre = di * agg + (di * di) * h_ref[...] + b_ref[...]
        h1 = jnp.maximum(pre, 0.0)
        hd2 = jnp.dot(h1, w_ref[...], preferred_element_type=jnp.float32)
        hd2_ref[...] = hd2
        g2_ref[...] = hd2 * di

    return pl.pallas_call(
        body,
        out_shape=[
            jax.ShapeDtypeStruct((n, h2), jnp.float32),
            jax.ShapeDtypeStruct((n, h2), jnp.float32),
        ],
    )(accp, h, dinv, b, W2)


def _tc3(accp, hd2, dinv, b, Wout, bout):
    n, _ = hd2.shape

    def body(accp_ref, hd2_ref, dinv_ref, b_ref, w_ref, bout_ref, out_ref):
        a = accp_ref[...]
        agg = a[0, :n] + a[1, :n]
        di = dinv_ref[...]
        pre = di * agg + (di * di) * hd2_ref[...] + b_ref[...]
        h2 = jnp.maximum(pre, 0.0)
        z = jnp.dot(h2, w_ref[...], preferred_element_type=jnp.float32) + bout_ref[...]
        out_ref[...] = 1.0 / (1.0 + jnp.exp(-z))

    return pl.pallas_call(
        body,
        out_shape=jax.ShapeDtypeStruct((n, 1), jnp.float32),
    )(accp, hd2, dinv, b, Wout, bout)


def kernel(x, edge_index, edge_weight, W1, b1, W2, b2, Wout, bout):
    e_total = edge_weight.shape[0]
    row = edge_index[0].astype(jnp.int32)
    col = edge_index[1].astype(jnp.int32)
    ew = edge_weight.astype(jnp.float32)

    chunk_total = NW * K * 4
    e_pad = ((e_total + chunk_total - 1) // chunk_total) * chunk_total
    pad = e_pad - e_total
    if pad:
        row = jnp.concatenate([row, jnp.zeros((pad,), jnp.int32)])
        col = jnp.concatenate([col, jnp.zeros((pad,), jnp.int32)])
        ew = jnp.concatenate([ew, jnp.zeros((pad,), jnp.float32)])
    n_chunks = e_pad // (NW * K)
    row = row.reshape(NW, n_chunks, K)
    col = col.reshape(NW, n_chunks, K)
    ew = ew.reshape(NW, n_chunks, K)

    h1n = W1.shape[1]
    h2n = W2.shape[1]
    zeros1 = jnp.zeros((N_PAD, L), jnp.float32)
    zeros_a = jnp.zeros((N_PAD, h1n), jnp.float32)
    zeros_b = jnp.zeros((N_PAD, h2n), jnp.float32)

    degp = _make_deg(n_chunks)(col, ew, zeros1)
    h, g1, dinv = _tc1(x, W1, degp)
    acc1 = _make_agg(h1n, n_chunks)(row, col, ew, g1, zeros_a)
    hd2, g2 = _tc2(acc1, h, dinv, b1.reshape(1, h1n), W2)
    acc2 = _make_agg(h2n, n_chunks)(row, col, ew, g2, zeros_b)
    return _tc3(acc2, hd2, dinv, b2.reshape(1, h2n), Wout, bout.reshape(1, 1))
